# baseline (speedup 1.0000x reference)
import jax
import jax.numpy as jnp
from jax import lax
from jax.experimental import pallas as pl
from jax.experimental.pallas import tpu as pltpu
from jax.experimental.pallas import tpu_sc as plsc

D_MODEL = 1024
NC = 2
NS = 16
NW = NC * NS

CHUNK = 8
NBUF = 12
NSB = 3


def _gather_kernel(table_hbm, idx_hbm, out_hbm, idx_v, rows_v, shared, gsem, xsem, ssem):
    b_per_w = idx_hbm.shape[0] // NW
    n_chunks = b_per_w // CHUNK
    sid = lax.axis_index("s")
    wid = sid * NC + lax.axis_index("c")
    base = wid * b_per_w
    pltpu.sync_copy(idx_hbm.at[pl.ds(base, b_per_w)], idx_v)

    def buf(m):
        return rows_v.at[pl.ds(m * CHUNK, CHUNK)]

    def slot(q):
        return shared.at[pl.ds((sid * NSB + q) * CHUNK, CHUNK)]

    def idxs(g):
        return idx_v.at[pl.ds(g * CHUNK, CHUNK)]

    def gather_copy(g, m):
        return pltpu.make_async_copy(table_hbm.at[idxs(g)], buf(m), gsem.at[m])

    def x_copy(m, q):
        return pltpu.make_async_copy(buf(m), slot(q), xsem.at[q])

    def drain_copy(g, q):
        return pltpu.make_async_copy(
            slot(q), out_hbm.at[pl.ds(base + g * CHUNK, CHUNK)], ssem.at[q]
        )

    for p in range(NBUF - 1):
        gather_copy(p, p).start()

    def body(g, carry):
        m = g % NBUF
        mp = (g + NBUF - 1) % NBUF
        q = g % NSB
        pm = (g + NBUF - 1) % NBUF
        pq = (g + NSB - 1) % NSB
        gather_copy(g, m).wait()
        pl.when(g >= NSB)(lambda: drain_copy(g - NSB, q).wait())
        x_copy(m, q).start()
        pl.when(g >= 1)(lambda: x_copy(pm, pq).wait())
        pl.when(g >= 1)(lambda: drain_copy(g - 1, pq).start())
        pl.when(g < n_chunks - (NBUF - 1))(
            lambda: gather_copy(g + NBUF - 1, mp).start()
        )
        return carry

    lax.fori_loop(0, n_chunks, body, 0)

    gl = n_chunks - 1
    x_copy(gl % NBUF, gl % NSB).wait()
    drain_copy(gl, gl % NSB).start()
    for j in range(n_chunks - NSB, n_chunks):
        drain_copy(j, j % NSB).wait()


def kernel(position_ids, table):
    batch, seq = position_ids.shape
    n = batch * seq
    b_per_w = n // NW
    idx_flat = position_ids.reshape(n).astype(jnp.int32)

    k = pl.kernel(
        _gather_kernel,
        out_type=jax.ShapeDtypeStruct((n, D_MODEL), jnp.float32),
        mesh=plsc.VectorSubcoreMesh(core_axis_name="c", subcore_axis_name="s"),
        scratch_types=[
            pltpu.VMEM((b_per_w,), jnp.int32),
            pltpu.VMEM((NBUF * CHUNK, D_MODEL), jnp.float32),
            pltpu.VMEM_SHARED((NS * NSB * CHUNK, D_MODEL), jnp.float32),
            pltpu.SemaphoreType.DMA((NBUF,)),
            pltpu.SemaphoreType.DMA((NSB,)),
            pltpu.SemaphoreType.DMA((NSB,)),
        ],
    )
    out = k(table, idx_flat)
    return out.reshape(batch, seq, D_MODEL)
